# baseline (speedup 1.0000x reference)
import functools

import jax
import jax.numpy as jnp
from jax import lax
from jax.experimental import pallas as pl
from jax.experimental.pallas import tpu as pltpu
from jax.experimental.pallas import tpu_sc as plsc

D_MODEL = 64
NB = 1
NBUF = 8


@functools.cache
def _build(batch: int, seq: int):
    info = plsc.get_sparse_core_info()
    nc, ns = info.num_cores, info.num_subcores
    nw = nc * ns
    assert batch % nw == 0
    rows_per_w = batch // nw
    assert rows_per_w % (NBUF * NB) == 0
    n_chunks = rows_per_w // NB
    n_groups = n_chunks // NBUF
    splits = []
    off = 0
    while off < seq:
        g = min(128, seq - off)
        splits.append((off, g))
        off += g
    mesh = plsc.VectorSubcoreMesh(core_axis_name="c", subcore_axis_name="s")

    @functools.partial(
        pl.kernel,
        mesh=mesh,
        out_type=jax.ShapeDtypeStruct((batch, seq, 128), jnp.float32),
        scratch_types=(
            [pltpu.VMEM((NB, seq), jnp.int32) for _ in range(NBUF)]
            + [pltpu.VMEM((NB, seq, D_MODEL), jnp.float32) for _ in range(NBUF)]
            + [pltpu.SemaphoreType.DMA for _ in range(2 * NBUF)]
        ),
        compiler_params=pltpu.CompilerParams(use_tc_tiling_on_sc=False),
    )
    def gather_kernel(idx_hbm, table_hbm, out_hbm, *scratch):
        idx_b = scratch[:NBUF]
        rows_b = scratch[NBUF:2 * NBUF]
        sem_g = scratch[2 * NBUF:3 * NBUF]
        sem_w = scratch[3 * NBUF:]
        wid = lax.axis_index("s") * nc + lax.axis_index("c")
        base = wid * rows_per_w

        def load_and_fire(c, s):
            b0 = base + c * NB
            pltpu.sync_copy(idx_hbm.at[pl.ds(b0, NB)], idx_b[s])
            for r in range(NB):
                for o, g in splits:
                    pltpu.async_copy(
                        table_hbm.at[idx_b[s].at[r, pl.ds(o, g)]],
                        rows_b[s].at[r, pl.ds(o, g)],
                        sem_g[s],
                    )

        def drain_g(s):
            pltpu.make_async_copy(
                out_hbm.at[pl.ds(0, NB), :, pl.ds(0, D_MODEL)], rows_b[s],
                sem_g[s]
            ).wait()

        def wb_start(c, s):
            b0 = base + c * NB
            pltpu.async_copy(
                rows_b[s], out_hbm.at[pl.ds(b0, NB), :, pl.ds(0, D_MODEL)],
                sem_w[s],
            )

        def drain_w(s):
            pltpu.make_async_copy(
                out_hbm.at[pl.ds(0, NB), :, pl.ds(0, D_MODEL)], rows_b[s],
                sem_w[s]
            ).wait()

        for s in range(NBUF - 1):
            load_and_fire(s, s)

        def body(i, carry):
            for s in range(NBUF):
                c = NBUF * i + s
                drain_g(s)
                wb_start(c, s)
                cf = c + NBUF - 1
                sf = (s + NBUF - 1) % NBUF

                @pl.when(cf < n_chunks)
                def _():
                    @pl.when(cf >= NBUF)
                    def _():
                        drain_w(sf)

                    load_and_fire(cf, sf)

            return carry

        lax.fori_loop(0, n_groups, body, 0)
        for s in range(NBUF):
            drain_w(s)

    return gather_kernel


def kernel(token_ids, weights):
    batch, seq = token_ids.shape
    padded = _build(batch, seq)(token_ids.astype(jnp.int32), weights)
    return padded[:, :, :D_MODEL]


# speedup vs baseline: 1.0028x; 1.0028x over previous
import functools

import jax
import jax.numpy as jnp
from jax import lax
from jax.experimental import pallas as pl
from jax.experimental.pallas import tpu as pltpu
from jax.experimental.pallas import tpu_sc as plsc

D_MODEL = 64
NB = 2
NBUF = 4


@functools.cache
def _build(batch: int, seq: int):
    info = plsc.get_sparse_core_info()
    nc, ns = info.num_cores, info.num_subcores
    nw = nc * ns
    assert batch % nw == 0
    rows_per_w = batch // nw
    assert rows_per_w % (NBUF * NB) == 0
    n_chunks = rows_per_w // NB
    n_groups = n_chunks // NBUF
    splits = []
    off = 0
    while off < seq:
        g = min(128, seq - off)
        splits.append((off, g))
        off += g
    mesh = plsc.VectorSubcoreMesh(core_axis_name="c", subcore_axis_name="s")

    @functools.partial(
        pl.kernel,
        mesh=mesh,
        out_type=jax.ShapeDtypeStruct((batch, seq, 128), jnp.float32),
        scratch_types=(
            [pltpu.VMEM((NB, seq), jnp.int32) for _ in range(NBUF)]
            + [pltpu.VMEM((NB, seq, D_MODEL), jnp.float32) for _ in range(NBUF)]
            + [pltpu.SemaphoreType.DMA for _ in range(2 * NBUF)]
        ),
        compiler_params=pltpu.CompilerParams(use_tc_tiling_on_sc=False),
    )
    def gather_kernel(idx_hbm, table_hbm, out_hbm, *scratch):
        idx_b = scratch[:NBUF]
        rows_b = scratch[NBUF:2 * NBUF]
        sem_g = scratch[2 * NBUF:3 * NBUF]
        sem_w = scratch[3 * NBUF:]
        wid = lax.axis_index("s") * nc + lax.axis_index("c")
        base = wid * rows_per_w

        def load_and_fire(c, s):
            b0 = base + c * NB
            pltpu.sync_copy(idx_hbm.at[pl.ds(b0, NB)], idx_b[s])
            for r in range(NB):
                for o, g in splits:
                    pltpu.async_copy(
                        table_hbm.at[idx_b[s].at[r, pl.ds(o, g)]],
                        rows_b[s].at[r, pl.ds(o, g)],
                        sem_g[s],
                    )

        def drain_g(s):
            pltpu.make_async_copy(
                out_hbm.at[pl.ds(0, NB), :, pl.ds(0, D_MODEL)], rows_b[s],
                sem_g[s]
            ).wait()

        def wb_start(c, s):
            b0 = base + c * NB
            pltpu.async_copy(
                rows_b[s], out_hbm.at[pl.ds(b0, NB), :, pl.ds(0, D_MODEL)],
                sem_w[s],
            )

        def drain_w(s):
            pltpu.make_async_copy(
                out_hbm.at[pl.ds(0, NB), :, pl.ds(0, D_MODEL)], rows_b[s],
                sem_w[s]
            ).wait()

        for s in range(NBUF - 1):
            load_and_fire(s, s)

        def body(i, carry):
            for s in range(NBUF):
                c = NBUF * i + s
                drain_g(s)
                wb_start(c, s)
                cf = c + NBUF - 1
                sf = (s + NBUF - 1) % NBUF

                @pl.when(cf < n_chunks)
                def _():
                    @pl.when(cf >= NBUF)
                    def _():
                        drain_w(sf)

                    load_and_fire(cf, sf)

            return carry

        lax.fori_loop(0, n_groups, body, 0)
        for s in range(NBUF):
            drain_w(s)

    return gather_kernel


def kernel(token_ids, weights):
    batch, seq = token_ids.shape
    padded = _build(batch, seq)(token_ids.astype(jnp.int32), weights)
    return padded[:, :, :D_MODEL]
